# baseline (speedup 1.0000x reference)
import jax
import jax.numpy as jnp
from jax import lax
from jax.experimental import pallas as pl
from jax.experimental.pallas import tpu as pltpu
from jax.experimental.pallas import tpu_sc as plsc

_EPS = 1e-12
_M = 128
_N = 10000
_E = 320000
_GAMMA = 0.8
_ITERS = 25

_NP = 10240
_NC, _NS = 2, 16
_NW = _NC * _NS
_EPW = _E // _NW
_CK = 80
_NCH = _EPW // _CK
_RPT = _NP // _NS
_NWB = _RPT // _CK
_LG = _M // 16


def _spmm_body(q_hbm, src_hbm, dst_hbm, w_hbm, out_hbm,
               src_a, dst_v0, dst_v1, dst_v2, w_v0, w_v1, w_v2,
               rows0, rows1, rows2, acc_s, sem_g0, sem_g1, sem_g2, sem_i):
    c = lax.axis_index("c")
    s = lax.axis_index("s")
    wid = s * _NC + c
    ebase = wid * _EPW
    row0 = s * _RPT
    zf = jnp.zeros((16,), jnp.float32)

    for r in range(_CK):
        for j in range(_LG):
            rows0[r, pl.ds(j * 16, 16)] = zf
    pltpu.make_async_copy(src_hbm.at[pl.ds(ebase, _EPW)], src_a, sem_g0).start()
    for b in range(_NWB):
        pltpu.make_async_copy(rows0, acc_s.at[pl.ds(row0 + b * _CK, _CK)],
                              sem_i).start()
    for b in range(_NWB):
        pltpu.make_async_copy(rows0, acc_s.at[pl.ds(row0 + b * _CK, _CK)],
                              sem_i).wait()
    pltpu.make_async_copy(src_hbm.at[pl.ds(ebase, _EPW)], src_a, sem_g0).wait()

    splat = [jnp.full((16,), i, jnp.int32) for i in range(16)]

    def issue(ci, dv, wv, rv, sg):
        off = ci * _CK
        pltpu.make_async_copy(dst_hbm.at[pl.ds(ebase + off, _CK)], dv, sg).start()
        pltpu.make_async_copy(w_hbm.at[pl.ds(ebase + off, _CK)], wv, sg).start()
        pltpu.make_async_copy(q_hbm.at[src_a.at[pl.ds(off, _CK)]], rv, sg).start()

    def wait_issue(ci, dv, wv, rv, sg):
        off = ci * _CK
        pltpu.make_async_copy(dst_hbm.at[pl.ds(ebase + off, _CK)], dv, sg).wait()
        pltpu.make_async_copy(w_hbm.at[pl.ds(ebase + off, _CK)], wv, sg).wait()
        pltpu.make_async_copy(q_hbm.at[src_a.at[pl.ds(off, _CK)]], rv, sg).wait()

    def process(dv, wv, rv):
        for g in range(_CK // 16):
            wvec = wv[pl.ds(g * 16, 16)]
            for i in range(16):
                wi = wvec.at[splat[i]].get(mode="promise_in_bounds")
                r = g * 16 + i
                for j in range(_LG):
                    rv[r, pl.ds(j * 16, 16)] = rv[r, pl.ds(j * 16, 16)] * wi
        pltpu.sync_copy(rv, acc_s.at[dv], add=True)

    bufs3 = ((dst_v0, w_v0, rows0, sem_g0),
             (dst_v1, w_v1, rows1, sem_g1),
             (dst_v2, w_v2, rows2, sem_g2))

    issue(0, *bufs3[0])
    issue(1, *bufs3[1])
    plsc.subcore_barrier()

    def body(ci, carry):
        for p in range(3):
            @pl.when(lax.rem(ci, 3) == p)
            def _(p=p):
                wait_issue(ci, *bufs3[p])

                @pl.when(ci <= _NCH - 3)
                def _():
                    issue(ci + 2, *bufs3[(p + 2) % 3])

                dv, wv, rv, _sg = bufs3[p]
                process(dv, wv, rv)

        return carry

    lax.fori_loop(0, _NCH, body, 0)

    plsc.subcore_barrier()

    bufs = (rows0, rows1)
    sems = (sem_g0, sem_g1)
    for b in range(_NWB):
        p = b & 1
        r0b = row0 + b * _CK
        if b >= 2:
            prev = row0 + (b - 2) * _CK
            pltpu.make_async_copy(bufs[p], out_hbm.at[pl.ds(c * _NP + prev, _CK)],
                                  sems[p]).wait()
        pltpu.sync_copy(acc_s.at[pl.ds(r0b, _CK)], bufs[p])
        pltpu.make_async_copy(bufs[p], out_hbm.at[pl.ds(c * _NP + r0b, _CK)],
                              sems[p]).start()
    for b in (_NWB - 2, _NWB - 1):
        p = b & 1
        r0b = row0 + b * _CK
        pltpu.make_async_copy(bufs[p], out_hbm.at[pl.ds(c * _NP + r0b, _CK)],
                              sems[p]).wait()


_spmm = pl.kernel(
    _spmm_body,
    out_type=jax.ShapeDtypeStruct((2 * _NP, _M), jnp.float32),
    mesh=plsc.VectorSubcoreMesh(core_axis_name="c", subcore_axis_name="s",
                                num_cores=_NC, num_subcores=_NS),
    scratch_types=[
        pltpu.VMEM((_EPW,), jnp.int32),
        pltpu.VMEM((_CK,), jnp.int32),
        pltpu.VMEM((_CK,), jnp.int32),
        pltpu.VMEM((_CK,), jnp.int32),
        pltpu.VMEM((_CK,), jnp.float32),
        pltpu.VMEM((_CK,), jnp.float32),
        pltpu.VMEM((_CK,), jnp.float32),
        pltpu.VMEM((_CK, _M), jnp.float32),
        pltpu.VMEM((_CK, _M), jnp.float32),
        pltpu.VMEM((_CK, _M), jnp.float32),
        pltpu.VMEM_SHARED((_NP, _M), jnp.float32),
        pltpu.SemaphoreType.DMA,
        pltpu.SemaphoreType.DMA,
        pltpu.SemaphoreType.DMA,
        pltpu.SemaphoreType.DMA,
    ],
)


def _gf_body(f_ref, g_ref):
    F = f_ref[...]
    FF = lax.dot_general(F, F, (((0,), (0,)), ((), ())),
                         preferred_element_type=jnp.float32)
    nrm = jnp.sqrt(jnp.sum(FF * FF))
    g_ref[...] = (1.0 / (nrm + _EPS)) * FF


_gf = pl.pallas_call(
    _gf_body,
    out_shape=jax.ShapeDtypeStruct((_M, _M), jnp.float32),
)

_NB = 2048


def _add_body(a_ref, o_ref):
    o_ref[...] = a_ref[0] + a_ref[1]


_combine = pl.pallas_call(
    _add_body,
    grid=(_NP // _NB,),
    in_specs=[pl.BlockSpec((2, _NB, _M), lambda i: (0, i, 0))],
    out_specs=pl.BlockSpec((_NB, _M), lambda i: (i, 0)),
    out_shape=jax.ShapeDtypeStruct((_NP, _M), jnp.float32),
)


def _upd_body(a_ref, g_ref, xt_ref, o_ref):
    p = a_ref[0] + a_ref[1]
    o_ref[...] = _GAMMA * lax.dot_general(
        p, g_ref[...], (((1,), (0,)), ((), ())),
        preferred_element_type=jnp.float32) + xt_ref[...]


_update = pl.pallas_call(
    _upd_body,
    grid=(_NP // _NB,),
    in_specs=[pl.BlockSpec((2, _NB, _M), lambda i: (0, i, 0)),
              pl.BlockSpec((_M, _M), lambda i: (0, 0)),
              pl.BlockSpec((_NB, _M), lambda i: (i, 0))],
    out_specs=pl.BlockSpec((_NB, _M), lambda i: (i, 0)),
    out_shape=jax.ShapeDtypeStruct((_NP, _M), jnp.float32),
)


def _fin_body(a_ref, g_ref, x_ref, o_ref):
    p = (a_ref[0] + a_ref[1])[:_N]
    o_ref[...] = _GAMMA * lax.dot_general(
        g_ref[...], p, (((1,), (1,)), ((), ())),
        preferred_element_type=jnp.float32) + x_ref[...]


_final = pl.pallas_call(
    _fin_body,
    out_shape=jax.ShapeDtypeStruct((_M, _N), jnp.float32),
)


def kernel(X, edge_index, edge_weight, F):
    src = edge_index[0]
    dst = edge_index[1]
    G = _gf(F)
    XT = jnp.pad(X.T, ((0, _NP - _N), (0, 0)))
    Q = XT
    Z = None
    for t in range(_ITERS - 1):
        A = _spmm(Q, src, dst, edge_weight).reshape(2, _NP, _M)
        R = _combine(A)
        B = _spmm(R, src, dst, edge_weight).reshape(2, _NP, _M)
        if t < _ITERS - 2:
            Q = _update(B, G, XT)
        else:
            Z = _final(B, G, X)
    return Z
